# baseline (speedup 1.0000x reference)
import functools

import jax
import jax.numpy as jnp
from jax import lax
from jax.experimental import pallas as pl
from jax.experimental.pallas import tpu as pltpu
from jax.experimental.pallas import tpu_sc as plsc

V, O, E, D = 8, 1250, 20000, 128
OP = 1256
GPC = 4
NT = 4
ETP = 5120
NCH = ETP // 128
K = 128
N_EDGES = float(V * E)
N_NODES = V * O


NBUF = 4
HR = 16


def _counts_body(s_hbm, o_hbm, zeros_hbm, outS_hbm, outD_hbm,
                 idx_sv, idx_ov, accS_loc, accD_loc, accS_sh, accD_sh):
    c = lax.axis_index("c")
    s = lax.axis_index("s")
    gl = s // NT
    q = lax.rem(s, NT)
    v = c * GPC + gl
    pltpu.sync_copy(s_hbm.at[v, q], idx_sv)
    pltpu.sync_copy(o_hbm.at[v, q], idx_ov)
    pltpu.sync_copy(zeros_hbm.at[pl.ds(0, HR)], accS_loc)
    pltpu.sync_copy(zeros_hbm.at[pl.ds(0, HR)], accD_loc)

    @pl.when(q == 0)
    def _():
        pltpu.sync_copy(zeros_hbm.at[pl.ds(0, HR)],
                        accS_sh.at[pl.ds(gl * HR, HR)])
        pltpu.sync_copy(zeros_hbm.at[pl.ds(0, HR)],
                        accD_sh.at[pl.ds(gl * HR, HR)])

    ones16 = jnp.ones((16,), jnp.float32)

    @pl.loop(0, ETP, step=16)
    def _(e):
        ixs = idx_sv[pl.ds(e, 16)]
        plsc.addupdate_scatter(
            accS_loc, [lax.shift_right_logical(ixs, 7), ixs & 127], ones16)
        ixo = idx_ov[pl.ds(e, 16)]
        plsc.addupdate_scatter(
            accD_loc, [lax.shift_right_logical(ixo, 7), ixo & 127], ones16)

    plsc.subcore_barrier()

    rows = gl * HR + lax.iota(jnp.int32, 16)
    pltpu.sync_copy(accS_loc, accS_sh.at[rows], add=True)
    pltpu.sync_copy(accD_loc, accD_sh.at[rows], add=True)

    plsc.subcore_barrier()

    @pl.when(q == 0)
    def _():
        pltpu.sync_copy(accS_sh.at[pl.ds(gl * HR, HR)], outS_hbm.at[v])
        pltpu.sync_copy(accD_sh.at[pl.ds(gl * HR, HR)], outD_hbm.at[v])


@functools.cache
def _get_counts():
    import dataclasses
    cp = pltpu.CompilerParams()
    if "needs_layout_passes" in pltpu.CompilerParams.__dataclass_fields__:
        cp = dataclasses.replace(cp, needs_layout_passes=False)
    return pl.kernel(
        _counts_body,
        out_type=(jax.ShapeDtypeStruct((V, HR, D), jnp.float32),
                  jax.ShapeDtypeStruct((V, HR, D), jnp.float32)),
        mesh=plsc.VectorSubcoreMesh(core_axis_name="c", subcore_axis_name="s"),
        compiler_params=cp,
        scratch_types=[pltpu.VMEM((ETP,), jnp.int32),
                       pltpu.VMEM((ETP,), jnp.int32),
                       pltpu.VMEM((HR, D), jnp.float32),
                       pltpu.VMEM((HR, D), jnp.float32),
                       pltpu.VMEM_SHARED((GPC * HR, D), jnp.float32),
                       pltpu.VMEM_SHARED((GPC * HR, D), jnp.float32)])


def _counts(*args):
    return _get_counts()(*args)


def _spool_body(b_hbm, sS_hbm, oG_hbm, zw_hbm, out_hbm,
                idx_s, idx_o, bufs, acc, gsem, ssem):
    c = lax.axis_index("c")
    s = lax.axis_index("s")
    gl = s // NT
    q = lax.rem(s, NT)
    v = c * GPC + gl
    pltpu.sync_copy(sS_hbm.at[v, q], idx_s)
    pltpu.sync_copy(oG_hbm.at[v, q], idx_o)

    for b in range(NBUF):
        pltpu.async_copy(b_hbm.at[idx_o.at[b]], bufs.at[b], gsem.at[b])

    @pl.when(q == 0)
    def _():
        pltpu.sync_copy(zw_hbm, acc.at[pl.ds(gl * OP, OP)])

    plsc.subcore_barrier()

    @pl.loop(0, NCH, step=NBUF)
    def _(ch0):
        for b in range(NBUF):
            ch = ch0 + b
            pltpu.make_async_copy(b_hbm.at[pl.ds(0, K)], bufs.at[b],
                                  gsem.at[b]).wait()
            pltpu.async_copy(bufs.at[b], acc.at[idx_s.at[ch]], ssem.at[b],
                             add=True)
        for b in range(NBUF):
            ch = ch0 + b
            pltpu.make_async_copy(b_hbm.at[pl.ds(0, K)], bufs.at[b],
                                  ssem.at[b]).wait()

            @pl.when(ch + NBUF < NCH)
            def _():
                pltpu.async_copy(b_hbm.at[idx_o.at[ch + NBUF]], bufs.at[b],
                                 gsem.at[b])

    plsc.subcore_barrier()

    @pl.when(q == 0)
    def _():
        pltpu.sync_copy(acc.at[pl.ds(gl * OP, OP)], out_hbm.at[v])


@functools.cache
def _get_spool():
    return pl.kernel(
        _spool_body,
        out_type=jax.ShapeDtypeStruct((V, OP, D), jnp.float32),
        mesh=plsc.VectorSubcoreMesh(core_axis_name="c", subcore_axis_name="s"),
        scratch_types=[pltpu.VMEM((NCH, K), jnp.int32),
                       pltpu.VMEM((NCH, K), jnp.int32),
                       pltpu.VMEM((NBUF, K, D), jnp.float32),
                       pltpu.VMEM_SHARED((GPC * OP, D), jnp.float32),
                       pltpu.SemaphoreType.DMA((NBUF,)),
                       pltpu.SemaphoreType.DMA((NBUF,))])


def _spool(*args):
    return _get_spool()(*args)


def _n2e_block(xf, c2, gamma2, beta2, w1t, w2t):
    w = jnp.dot(c2, xf, preferred_element_type=jnp.float32)
    w2 = jnp.dot(c2, xf * xf, preferred_element_type=jnp.float32)
    mean = w * (1.0 / N_EDGES)
    var = w2 * (1.0 / N_EDGES) - mean * mean
    scale = gamma2 * lax.rsqrt(var + 1e-5)
    shift = beta2 - mean * scale
    xs = jnp.maximum(xf * scale[0:1, :] + shift[0:1, :], 0.0)
    xd = jnp.maximum(xf * scale[1:2, :] + shift[1:2, :], 0.0)
    a = jnp.dot(xs, w1t, preferred_element_type=jnp.float32)
    b = jnp.dot(xd, w2t, preferred_element_type=jnp.float32)
    return a, b


def _e2n_block(a, s_sum, cs, b1, gamma, beta, w3t, b3):
    pool = (cs * (a + b1) + s_sum) / jnp.maximum(cs, 1.0)
    m = jnp.sum(pool, axis=0, keepdims=True) * (1.0 / N_NODES)
    var = jnp.sum(pool * pool, axis=0, keepdims=True) * (1.0 / N_NODES) - m * m
    scale = gamma * lax.rsqrt(var + 1e-5)
    shift = beta - m * scale
    xn = jnp.maximum(pool * scale + shift, 0.0)
    return jnp.dot(xn, w3t, preferred_element_type=jnp.float32) + b3


def _tc_a_body(x_ref, c2_ref, gamma_ref, beta_ref, w1t_ref, w2t_ref,
               a_ref, bm_ref):
    a, b = _n2e_block(x_ref[...], c2_ref[...], gamma_ref[...], beta_ref[...],
                      w1t_ref[...], w2t_ref[...])
    a_ref[...] = a
    bm_ref[...] = b


def _tc_b_body(a0_ref, s0_ref, cs_ref, b1_ref, g2_ref, bt2_ref, w3t_ref,
               b3_ref, c2_ref, gamma1_ref, beta1_ref, w1t_ref, w2t_ref,
               a1_ref, bm1_ref):
    x1 = _e2n_block(a0_ref[...], s0_ref[...], cs_ref[...], b1_ref[...],
                    g2_ref[...], bt2_ref[...], w3t_ref[...], b3_ref[...])
    a, b = _n2e_block(x1, c2_ref[...], gamma1_ref[...], beta1_ref[...],
                      w1t_ref[...], w2t_ref[...])
    a1_ref[...] = a
    bm1_ref[...] = b


def _tc_c_body(a1_ref, s1_ref, cs_ref, b1_ref, g2_ref, bt2_ref, w3t_ref,
               b3_ref, xin_ref, out_ref):
    x2 = _e2n_block(a1_ref[...], s1_ref[...], cs_ref[...], b1_ref[...],
                    g2_ref[...], bt2_ref[...], w3t_ref[...], b3_ref[...])
    out_ref[...] = x2 + xin_ref[...]


_f32 = jnp.float32
_tc_a = pl.pallas_call(
    _tc_a_body,
    out_shape=(jax.ShapeDtypeStruct((N_NODES, D), _f32),
               jax.ShapeDtypeStruct((N_NODES, D), _f32)))
_tc_b = pl.pallas_call(
    _tc_b_body,
    out_shape=(jax.ShapeDtypeStruct((N_NODES, D), _f32),
               jax.ShapeDtypeStruct((N_NODES, D), _f32)))
_tc_c = pl.pallas_call(
    _tc_c_body,
    out_shape=jax.ShapeDtypeStruct((N_NODES, D), _f32))


def kernel(input, edges,
           g0_n2e_gamma, g0_n2e_beta, g0_n2e_W, g0_n2e_b,
           g0_e2n_gamma, g0_e2n_beta, g0_e2n_W, g0_e2n_b,
           g1_n2e_gamma, g1_n2e_beta, g1_n2e_W, g1_n2e_b,
           g1_e2n_gamma, g1_e2n_beta, g1_e2n_W, g1_e2n_b):
    x0 = input.reshape(N_NODES, D)
    s_idx = edges[:, :, 0]
    o_idx = edges[:, :, 2]

    pad = NT * ETP - E
    glv = (jnp.arange(V, dtype=jnp.int32) % GPC)[:, None] * OP
    voff = jnp.arange(V, dtype=jnp.int32)[:, None] * O
    s_pad = jnp.concatenate(
        [s_idx, jnp.full((V, pad), O, jnp.int32)], axis=1)
    o_pad0 = jnp.concatenate(
        [o_idx, jnp.zeros((V, pad), jnp.int32)], axis=1)
    o_padO = jnp.concatenate(
        [o_idx, jnp.full((V, pad), O, jnp.int32)], axis=1)
    sS = (s_pad + glv).reshape(V, NT, NCH, K)
    oG = (o_pad0 + voff).reshape(V, NT, NCH, K)
    s_flat = s_pad.reshape(V, NT, ETP)
    o_flat = o_padO.reshape(V, NT, ETP)
    zerosD = jnp.zeros((OP, D), _f32)

    cnt_s3, cnt_d3 = _counts(s_flat, o_flat, zerosD)
    cs = cnt_s3.reshape(V, HR * D)[:, :O].reshape(N_NODES)
    cd = cnt_d3.reshape(V, HR * D)[:, :O].reshape(N_NODES)
    c2 = jnp.stack([cs, cd])
    cs_col = cs.reshape(N_NODES, 1)

    a0, b0m = _tc_a(
        x0, c2,
        g0_n2e_gamma.reshape(2, D), g0_n2e_beta.reshape(2, D),
        g0_n2e_W[:, :D].T, g0_n2e_W[:, D:].T)

    s0 = _spool(b0m, sS, oG, zerosD)[:, :O, :].reshape(N_NODES, D)

    a1, b1m = _tc_b(
        a0, s0, cs_col, g0_n2e_b.reshape(1, D),
        g0_e2n_gamma.reshape(1, D), g0_e2n_beta.reshape(1, D),
        g0_e2n_W.T, g0_e2n_b.reshape(1, D),
        c2,
        g1_n2e_gamma.reshape(2, D), g1_n2e_beta.reshape(2, D),
        g1_n2e_W[:, :D].T, g1_n2e_W[:, D:].T)

    s1 = _spool(b1m, sS, oG, zerosD)[:, :O, :].reshape(N_NODES, D)

    out = _tc_c(
        a1, s1, cs_col, g1_n2e_b.reshape(1, D),
        g1_e2n_gamma.reshape(1, D), g1_e2n_beta.reshape(1, D),
        g1_e2n_W.T, g1_e2n_b.reshape(1, D),
        x0)
    return out.reshape(V, O, D)


# speedup vs baseline: 1.0007x; 1.0007x over previous
import functools

import jax
import jax.numpy as jnp
from jax import lax
from jax.experimental import pallas as pl
from jax.experimental.pallas import tpu as pltpu
from jax.experimental.pallas import tpu_sc as plsc

V, O, E, D = 8, 1250, 20000, 128
OP = 1256
GPC = 4
NT = 4
ETP = 5120
NCH = ETP // 128
K = 128
N_EDGES = float(V * E)
N_NODES = V * O


NBUF = 4
HR = 16


def _counts_body(s_hbm, o_hbm, zeros_hbm, outS_hbm, outD_hbm,
                 idx_sv, idx_ov, accS_loc, accD_loc, accS_sh, accD_sh):
    c = lax.axis_index("c")
    s = lax.axis_index("s")
    gl = s // NT
    q = lax.rem(s, NT)
    v = c * GPC + gl
    pltpu.sync_copy(s_hbm.at[v, q], idx_sv)
    pltpu.sync_copy(o_hbm.at[v, q], idx_ov)
    pltpu.sync_copy(zeros_hbm.at[pl.ds(0, HR)], accS_loc)
    pltpu.sync_copy(zeros_hbm.at[pl.ds(0, HR)], accD_loc)

    @pl.when(q == 0)
    def _():
        pltpu.sync_copy(zeros_hbm.at[pl.ds(0, HR)],
                        accS_sh.at[pl.ds(gl * HR, HR)])
        pltpu.sync_copy(zeros_hbm.at[pl.ds(0, HR)],
                        accD_sh.at[pl.ds(gl * HR, HR)])

    ones16 = jnp.ones((16,), jnp.float32)

    @pl.loop(0, ETP, step=16)
    def _(e):
        ixs = idx_sv[pl.ds(e, 16)]
        plsc.addupdate_scatter(
            accS_loc, [lax.shift_right_logical(ixs, 7), ixs & 127], ones16)
        ixo = idx_ov[pl.ds(e, 16)]
        plsc.addupdate_scatter(
            accD_loc, [lax.shift_right_logical(ixo, 7), ixo & 127], ones16)

    plsc.subcore_barrier()

    rows = gl * HR + lax.iota(jnp.int32, 16)
    pltpu.sync_copy(accS_loc, accS_sh.at[rows], add=True)
    pltpu.sync_copy(accD_loc, accD_sh.at[rows], add=True)

    plsc.subcore_barrier()

    @pl.when(q == 0)
    def _():
        pltpu.sync_copy(accS_sh.at[pl.ds(gl * HR, HR)], outS_hbm.at[v])
        pltpu.sync_copy(accD_sh.at[pl.ds(gl * HR, HR)], outD_hbm.at[v])


@functools.cache
def _get_counts():
    import dataclasses
    cp = pltpu.CompilerParams()
    if "needs_layout_passes" in pltpu.CompilerParams.__dataclass_fields__:
        cp = dataclasses.replace(cp, needs_layout_passes=False)
    return pl.kernel(
        _counts_body,
        out_type=(jax.ShapeDtypeStruct((V, HR, D), jnp.float32),
                  jax.ShapeDtypeStruct((V, HR, D), jnp.float32)),
        mesh=plsc.VectorSubcoreMesh(core_axis_name="c", subcore_axis_name="s"),
        compiler_params=cp,
        scratch_types=[pltpu.VMEM((ETP,), jnp.int32),
                       pltpu.VMEM((ETP,), jnp.int32),
                       pltpu.VMEM((HR, D), jnp.float32),
                       pltpu.VMEM((HR, D), jnp.float32),
                       pltpu.VMEM_SHARED((GPC * HR, D), jnp.float32),
                       pltpu.VMEM_SHARED((GPC * HR, D), jnp.float32)])


def _counts(*args):
    return _get_counts()(*args)


def _spool_body(b_hbm, sS_hbm, oG_hbm, zw_hbm, out_hbm,
                idx_s, idx_o, bufs, acc, gsem, ssem):
    c = lax.axis_index("c")
    s = lax.axis_index("s")
    gl = s // NT
    q = lax.rem(s, NT)
    v = c * GPC + gl
    pltpu.sync_copy(sS_hbm.at[v, q], idx_s)
    pltpu.sync_copy(oG_hbm.at[v, q], idx_o)

    for b in range(NBUF):
        pltpu.async_copy(b_hbm.at[idx_o.at[b]], bufs.at[b], gsem.at[b])

    @pl.when(q == 0)
    def _():
        pltpu.sync_copy(zw_hbm, acc.at[pl.ds(gl * OP, OP)])

    plsc.subcore_barrier()

    @pl.loop(0, NCH, step=NBUF)
    def _(ch0):
        for b in range(NBUF):
            ch = ch0 + b
            pltpu.make_async_copy(b_hbm.at[pl.ds(0, K)], bufs.at[b],
                                  gsem.at[b]).wait()
            pltpu.async_copy(bufs.at[b], acc.at[idx_s.at[ch]], ssem.at[b],
                             add=True)
        for b in range(NBUF):
            ch = ch0 + b
            pltpu.make_async_copy(b_hbm.at[pl.ds(0, K)], bufs.at[b],
                                  ssem.at[b]).wait()

            @pl.when(ch + NBUF < NCH)
            def _():
                pltpu.async_copy(b_hbm.at[idx_o.at[ch + NBUF]], bufs.at[b],
                                 gsem.at[b])

    plsc.subcore_barrier()

    @pl.when(q == 0)
    def _():
        pltpu.sync_copy(acc.at[pl.ds(gl * OP, O)], out_hbm.at[v])


@functools.cache
def _get_spool():
    return pl.kernel(
        _spool_body,
        out_type=jax.ShapeDtypeStruct((V, O, D), jnp.float32),
        mesh=plsc.VectorSubcoreMesh(core_axis_name="c", subcore_axis_name="s"),
        scratch_types=[pltpu.VMEM((NCH, K), jnp.int32),
                       pltpu.VMEM((NCH, K), jnp.int32),
                       pltpu.VMEM((NBUF, K, D), jnp.float32),
                       pltpu.VMEM_SHARED((GPC * OP, D), jnp.float32),
                       pltpu.SemaphoreType.DMA((NBUF,)),
                       pltpu.SemaphoreType.DMA((NBUF,))])


def _spool(*args):
    return _get_spool()(*args)


def _n2e_block(xf, c2, gamma2, beta2, w1t, w2t):
    w = jnp.dot(c2, xf, preferred_element_type=jnp.float32)
    w2 = jnp.dot(c2, xf * xf, preferred_element_type=jnp.float32)
    mean = w * (1.0 / N_EDGES)
    var = w2 * (1.0 / N_EDGES) - mean * mean
    scale = gamma2 * lax.rsqrt(var + 1e-5)
    shift = beta2 - mean * scale
    xs = jnp.maximum(xf * scale[0:1, :] + shift[0:1, :], 0.0)
    xd = jnp.maximum(xf * scale[1:2, :] + shift[1:2, :], 0.0)
    a = jnp.dot(xs, w1t, preferred_element_type=jnp.float32)
    b = jnp.dot(xd, w2t, preferred_element_type=jnp.float32)
    return a, b


def _e2n_block(a, s_sum, cs, b1, gamma, beta, w3t, b3):
    pool = (cs * (a + b1) + s_sum) / jnp.maximum(cs, 1.0)
    m = jnp.sum(pool, axis=0, keepdims=True) * (1.0 / N_NODES)
    var = jnp.sum(pool * pool, axis=0, keepdims=True) * (1.0 / N_NODES) - m * m
    scale = gamma * lax.rsqrt(var + 1e-5)
    shift = beta - m * scale
    xn = jnp.maximum(pool * scale + shift, 0.0)
    return jnp.dot(xn, w3t, preferred_element_type=jnp.float32) + b3


def _tc_a_body(x_ref, c2_ref, gamma_ref, beta_ref, w1t_ref, w2t_ref,
               a_ref, bm_ref):
    a, b = _n2e_block(x_ref[...], c2_ref[...], gamma_ref[...], beta_ref[...],
                      w1t_ref[...], w2t_ref[...])
    a_ref[...] = a
    bm_ref[...] = b


def _tc_b_body(a0_ref, s0_ref, cs_ref, b1_ref, g2_ref, bt2_ref, w3t_ref,
               b3_ref, c2_ref, gamma1_ref, beta1_ref, w1t_ref, w2t_ref,
               a1_ref, bm1_ref):
    x1 = _e2n_block(a0_ref[...], s0_ref[...], cs_ref[...], b1_ref[...],
                    g2_ref[...], bt2_ref[...], w3t_ref[...], b3_ref[...])
    a, b = _n2e_block(x1, c2_ref[...], gamma1_ref[...], beta1_ref[...],
                      w1t_ref[...], w2t_ref[...])
    a1_ref[...] = a
    bm1_ref[...] = b


def _tc_c_body(a1_ref, s1_ref, cs_ref, b1_ref, g2_ref, bt2_ref, w3t_ref,
               b3_ref, xin_ref, out_ref):
    x2 = _e2n_block(a1_ref[...], s1_ref[...], cs_ref[...], b1_ref[...],
                    g2_ref[...], bt2_ref[...], w3t_ref[...], b3_ref[...])
    out_ref[...] = x2 + xin_ref[...]


_f32 = jnp.float32
_tc_a = pl.pallas_call(
    _tc_a_body,
    out_shape=(jax.ShapeDtypeStruct((N_NODES, D), _f32),
               jax.ShapeDtypeStruct((N_NODES, D), _f32)))
_tc_b = pl.pallas_call(
    _tc_b_body,
    out_shape=(jax.ShapeDtypeStruct((N_NODES, D), _f32),
               jax.ShapeDtypeStruct((N_NODES, D), _f32)))
_tc_c = pl.pallas_call(
    _tc_c_body,
    out_shape=jax.ShapeDtypeStruct((N_NODES, D), _f32))


def kernel(input, edges,
           g0_n2e_gamma, g0_n2e_beta, g0_n2e_W, g0_n2e_b,
           g0_e2n_gamma, g0_e2n_beta, g0_e2n_W, g0_e2n_b,
           g1_n2e_gamma, g1_n2e_beta, g1_n2e_W, g1_n2e_b,
           g1_e2n_gamma, g1_e2n_beta, g1_e2n_W, g1_e2n_b):
    x0 = input.reshape(N_NODES, D)
    s_idx = edges[:, :, 0]
    o_idx = edges[:, :, 2]

    pad = NT * ETP - E
    glv = (jnp.arange(V, dtype=jnp.int32) % GPC)[:, None] * OP
    voff = jnp.arange(V, dtype=jnp.int32)[:, None] * O
    s_pad = jnp.concatenate(
        [s_idx, jnp.full((V, pad), O, jnp.int32)], axis=1)
    o_pad0 = jnp.concatenate(
        [o_idx, jnp.zeros((V, pad), jnp.int32)], axis=1)
    o_padO = jnp.concatenate(
        [o_idx, jnp.full((V, pad), O, jnp.int32)], axis=1)
    sS = (s_pad + glv).reshape(V, NT, NCH, K)
    oG = (o_pad0 + voff).reshape(V, NT, NCH, K)
    s_flat = s_pad.reshape(V, NT, ETP)
    o_flat = o_padO.reshape(V, NT, ETP)
    zerosD = jnp.zeros((OP, D), _f32)

    cnt_s3, cnt_d3 = _counts(s_flat, o_flat, zerosD)
    cs = cnt_s3.reshape(V, HR * D)[:, :O].reshape(N_NODES)
    cd = cnt_d3.reshape(V, HR * D)[:, :O].reshape(N_NODES)
    c2 = jnp.stack([cs, cd])
    cs_col = cs.reshape(N_NODES, 1)

    a0, b0m = _tc_a(
        x0, c2,
        g0_n2e_gamma.reshape(2, D), g0_n2e_beta.reshape(2, D),
        g0_n2e_W[:, :D].T, g0_n2e_W[:, D:].T)

    s0 = _spool(b0m, sS, oG, zerosD).reshape(N_NODES, D)

    a1, b1m = _tc_b(
        a0, s0, cs_col, g0_n2e_b.reshape(1, D),
        g0_e2n_gamma.reshape(1, D), g0_e2n_beta.reshape(1, D),
        g0_e2n_W.T, g0_e2n_b.reshape(1, D),
        c2,
        g1_n2e_gamma.reshape(2, D), g1_n2e_beta.reshape(2, D),
        g1_n2e_W[:, :D].T, g1_n2e_W[:, D:].T)

    s1 = _spool(b1m, sS, oG, zerosD).reshape(N_NODES, D)

    out = _tc_c(
        a1, s1, cs_col, g1_n2e_b.reshape(1, D),
        g1_e2n_gamma.reshape(1, D), g1_e2n_beta.reshape(1, D),
        g1_e2n_W.T, g1_e2n_b.reshape(1, D),
        x0)
    return out.reshape(V, O, D)


# speedup vs baseline: 1.0144x; 1.0136x over previous
import functools

import jax
import jax.numpy as jnp
from jax import lax
from jax.experimental import pallas as pl
from jax.experimental.pallas import tpu as pltpu
from jax.experimental.pallas import tpu_sc as plsc

V, O, E, D = 8, 1250, 20000, 128
OP = 1256
GPC = 4
NT = 4
ETP = 5120
K = 64
NCH = ETP // K
N_EDGES = float(V * E)
N_NODES = V * O


NBUF = 8
HR = 16


def _counts_body(s_hbm, o_hbm, zeros_hbm, outS_hbm, outD_hbm,
                 idx_sv, idx_ov, accS_loc, accD_loc, accS_sh, accD_sh):
    c = lax.axis_index("c")
    s = lax.axis_index("s")
    gl = s // NT
    q = lax.rem(s, NT)
    v = c * GPC + gl
    pltpu.sync_copy(s_hbm.at[v, q], idx_sv)
    pltpu.sync_copy(o_hbm.at[v, q], idx_ov)
    pltpu.sync_copy(zeros_hbm.at[pl.ds(0, HR)], accS_loc)
    pltpu.sync_copy(zeros_hbm.at[pl.ds(0, HR)], accD_loc)

    @pl.when(q == 0)
    def _():
        pltpu.sync_copy(zeros_hbm.at[pl.ds(0, HR)],
                        accS_sh.at[pl.ds(gl * HR, HR)])
        pltpu.sync_copy(zeros_hbm.at[pl.ds(0, HR)],
                        accD_sh.at[pl.ds(gl * HR, HR)])

    ones16 = jnp.ones((16,), jnp.float32)

    @pl.loop(0, ETP, step=16)
    def _(e):
        ixs = idx_sv[pl.ds(e, 16)]
        plsc.addupdate_scatter(
            accS_loc, [lax.shift_right_logical(ixs, 7), ixs & 127], ones16)
        ixo = idx_ov[pl.ds(e, 16)]
        plsc.addupdate_scatter(
            accD_loc, [lax.shift_right_logical(ixo, 7), ixo & 127], ones16)

    plsc.subcore_barrier()

    rows = gl * HR + lax.iota(jnp.int32, 16)
    pltpu.sync_copy(accS_loc, accS_sh.at[rows], add=True)
    pltpu.sync_copy(accD_loc, accD_sh.at[rows], add=True)

    plsc.subcore_barrier()

    @pl.when(q == 0)
    def _():
        pltpu.sync_copy(accS_sh.at[pl.ds(gl * HR, HR)], outS_hbm.at[v])
        pltpu.sync_copy(accD_sh.at[pl.ds(gl * HR, HR)], outD_hbm.at[v])


@functools.cache
def _get_counts():
    import dataclasses
    cp = pltpu.CompilerParams()
    if "needs_layout_passes" in pltpu.CompilerParams.__dataclass_fields__:
        cp = dataclasses.replace(cp, needs_layout_passes=False)
    return pl.kernel(
        _counts_body,
        out_type=(jax.ShapeDtypeStruct((V, HR, D), jnp.float32),
                  jax.ShapeDtypeStruct((V, HR, D), jnp.float32)),
        mesh=plsc.VectorSubcoreMesh(core_axis_name="c", subcore_axis_name="s"),
        compiler_params=cp,
        scratch_types=[pltpu.VMEM((ETP,), jnp.int32),
                       pltpu.VMEM((ETP,), jnp.int32),
                       pltpu.VMEM((HR, D), jnp.float32),
                       pltpu.VMEM((HR, D), jnp.float32),
                       pltpu.VMEM_SHARED((GPC * HR, D), jnp.float32),
                       pltpu.VMEM_SHARED((GPC * HR, D), jnp.float32)])


def _counts(*args):
    return _get_counts()(*args)


def _spool_body(b_hbm, sS_hbm, oG_hbm, zw_hbm, out_hbm,
                idx_s, idx_o, bufs, acc, gsem, ssem):
    c = lax.axis_index("c")
    s = lax.axis_index("s")
    gl = s // NT
    q = lax.rem(s, NT)
    v = c * GPC + gl
    pltpu.sync_copy(sS_hbm.at[v, q], idx_s)
    pltpu.sync_copy(oG_hbm.at[v, q], idx_o)

    for b in range(NBUF):
        pltpu.async_copy(b_hbm.at[idx_o.at[b]], bufs.at[b], gsem.at[b])

    @pl.when(q == 0)
    def _():
        pltpu.sync_copy(zw_hbm, acc.at[pl.ds(gl * OP, OP)])

    plsc.subcore_barrier()

    @pl.loop(0, NCH, step=NBUF)
    def _(ch0):
        for b in range(NBUF):
            ch = ch0 + b
            pltpu.make_async_copy(b_hbm.at[pl.ds(0, K)], bufs.at[b],
                                  gsem.at[b]).wait()
            pltpu.async_copy(bufs.at[b], acc.at[idx_s.at[ch]], ssem.at[b],
                             add=True)
        for b in range(NBUF):
            ch = ch0 + b
            pltpu.make_async_copy(b_hbm.at[pl.ds(0, K)], bufs.at[b],
                                  ssem.at[b]).wait()

            @pl.when(ch + NBUF < NCH)
            def _():
                pltpu.async_copy(b_hbm.at[idx_o.at[ch + NBUF]], bufs.at[b],
                                 gsem.at[b])

    plsc.subcore_barrier()

    @pl.when(q == 0)
    def _():
        pltpu.sync_copy(acc.at[pl.ds(gl * OP, O)], out_hbm.at[v])


@functools.cache
def _get_spool():
    return pl.kernel(
        _spool_body,
        out_type=jax.ShapeDtypeStruct((V, O, D), jnp.float32),
        mesh=plsc.VectorSubcoreMesh(core_axis_name="c", subcore_axis_name="s"),
        scratch_types=[pltpu.VMEM((NCH, K), jnp.int32),
                       pltpu.VMEM((NCH, K), jnp.int32),
                       pltpu.VMEM((NBUF, K, D), jnp.float32),
                       pltpu.VMEM_SHARED((GPC * OP, D), jnp.float32),
                       pltpu.SemaphoreType.DMA((NBUF,)),
                       pltpu.SemaphoreType.DMA((NBUF,))])


def _spool(*args):
    return _get_spool()(*args)


def _n2e_block(xf, c2, gamma2, beta2, w1t, w2t):
    w = jnp.dot(c2, xf, preferred_element_type=jnp.float32)
    w2 = jnp.dot(c2, xf * xf, preferred_element_type=jnp.float32)
    mean = w * (1.0 / N_EDGES)
    var = w2 * (1.0 / N_EDGES) - mean * mean
    scale = gamma2 * lax.rsqrt(var + 1e-5)
    shift = beta2 - mean * scale
    xs = jnp.maximum(xf * scale[0:1, :] + shift[0:1, :], 0.0)
    xd = jnp.maximum(xf * scale[1:2, :] + shift[1:2, :], 0.0)
    a = jnp.dot(xs, w1t, preferred_element_type=jnp.float32)
    b = jnp.dot(xd, w2t, preferred_element_type=jnp.float32)
    return a, b


def _e2n_block(a, s_sum, cs, b1, gamma, beta, w3t, b3):
    pool = (cs * (a + b1) + s_sum) / jnp.maximum(cs, 1.0)
    m = jnp.sum(pool, axis=0, keepdims=True) * (1.0 / N_NODES)
    var = jnp.sum(pool * pool, axis=0, keepdims=True) * (1.0 / N_NODES) - m * m
    scale = gamma * lax.rsqrt(var + 1e-5)
    shift = beta - m * scale
    xn = jnp.maximum(pool * scale + shift, 0.0)
    return jnp.dot(xn, w3t, preferred_element_type=jnp.float32) + b3


def _tc_a_body(x_ref, c2_ref, gamma_ref, beta_ref, w1t_ref, w2t_ref,
               a_ref, bm_ref):
    a, b = _n2e_block(x_ref[...], c2_ref[...], gamma_ref[...], beta_ref[...],
                      w1t_ref[...], w2t_ref[...])
    a_ref[...] = a
    bm_ref[...] = b


def _tc_b_body(a0_ref, s0_ref, cs_ref, b1_ref, g2_ref, bt2_ref, w3t_ref,
               b3_ref, c2_ref, gamma1_ref, beta1_ref, w1t_ref, w2t_ref,
               a1_ref, bm1_ref):
    x1 = _e2n_block(a0_ref[...], s0_ref[...], cs_ref[...], b1_ref[...],
                    g2_ref[...], bt2_ref[...], w3t_ref[...], b3_ref[...])
    a, b = _n2e_block(x1, c2_ref[...], gamma1_ref[...], beta1_ref[...],
                      w1t_ref[...], w2t_ref[...])
    a1_ref[...] = a
    bm1_ref[...] = b


def _tc_c_body(a1_ref, s1_ref, cs_ref, b1_ref, g2_ref, bt2_ref, w3t_ref,
               b3_ref, xin_ref, out_ref):
    x2 = _e2n_block(a1_ref[...], s1_ref[...], cs_ref[...], b1_ref[...],
                    g2_ref[...], bt2_ref[...], w3t_ref[...], b3_ref[...])
    out_ref[...] = x2 + xin_ref[...]


_f32 = jnp.float32
_tc_a = pl.pallas_call(
    _tc_a_body,
    out_shape=(jax.ShapeDtypeStruct((N_NODES, D), _f32),
               jax.ShapeDtypeStruct((N_NODES, D), _f32)))
_tc_b = pl.pallas_call(
    _tc_b_body,
    out_shape=(jax.ShapeDtypeStruct((N_NODES, D), _f32),
               jax.ShapeDtypeStruct((N_NODES, D), _f32)))
_tc_c = pl.pallas_call(
    _tc_c_body,
    out_shape=jax.ShapeDtypeStruct((N_NODES, D), _f32))


def kernel(input, edges,
           g0_n2e_gamma, g0_n2e_beta, g0_n2e_W, g0_n2e_b,
           g0_e2n_gamma, g0_e2n_beta, g0_e2n_W, g0_e2n_b,
           g1_n2e_gamma, g1_n2e_beta, g1_n2e_W, g1_n2e_b,
           g1_e2n_gamma, g1_e2n_beta, g1_e2n_W, g1_e2n_b):
    x0 = input.reshape(N_NODES, D)
    s_idx = edges[:, :, 0]
    o_idx = edges[:, :, 2]

    pad = NT * ETP - E
    glv = (jnp.arange(V, dtype=jnp.int32) % GPC)[:, None] * OP
    voff = jnp.arange(V, dtype=jnp.int32)[:, None] * O
    s_pad = jnp.concatenate(
        [s_idx, jnp.full((V, pad), O, jnp.int32)], axis=1)
    o_pad0 = jnp.concatenate(
        [o_idx, jnp.zeros((V, pad), jnp.int32)], axis=1)
    o_padO = jnp.concatenate(
        [o_idx, jnp.full((V, pad), O, jnp.int32)], axis=1)
    sS = (s_pad + glv).reshape(V, NT, NCH, K)
    oG = (o_pad0 + voff).reshape(V, NT, NCH, K)
    s_flat = s_pad.reshape(V, NT, ETP)
    o_flat = o_padO.reshape(V, NT, ETP)
    zerosD = jnp.zeros((OP, D), _f32)

    cnt_s3, cnt_d3 = _counts(s_flat, o_flat, zerosD)
    cs = cnt_s3.reshape(V, HR * D)[:, :O].reshape(N_NODES)
    cd = cnt_d3.reshape(V, HR * D)[:, :O].reshape(N_NODES)
    c2 = jnp.stack([cs, cd])
    cs_col = cs.reshape(N_NODES, 1)

    a0, b0m = _tc_a(
        x0, c2,
        g0_n2e_gamma.reshape(2, D), g0_n2e_beta.reshape(2, D),
        g0_n2e_W[:, :D].T, g0_n2e_W[:, D:].T)

    s0 = _spool(b0m, sS, oG, zerosD).reshape(N_NODES, D)

    a1, b1m = _tc_b(
        a0, s0, cs_col, g0_n2e_b.reshape(1, D),
        g0_e2n_gamma.reshape(1, D), g0_e2n_beta.reshape(1, D),
        g0_e2n_W.T, g0_e2n_b.reshape(1, D),
        c2,
        g1_n2e_gamma.reshape(2, D), g1_n2e_beta.reshape(2, D),
        g1_n2e_W[:, :D].T, g1_n2e_W[:, D:].T)

    s1 = _spool(b1m, sS, oG, zerosD).reshape(N_NODES, D)

    out = _tc_c(
        a1, s1, cs_col, g1_n2e_b.reshape(1, D),
        g1_e2n_gamma.reshape(1, D), g1_e2n_beta.reshape(1, D),
        g1_e2n_W.T, g1_e2n_b.reshape(1, D),
        x0)
    return out.reshape(V, O, D)
